# Initial kernel scaffold; baseline (speedup 1.0000x reference)
import jax
import jax.numpy as jnp
from jax.experimental import pallas as pl


def kernel(pred_poly, pred_prob, gt_py, gt_num):
    raise NotImplementedError("write your pallas kernel here")


# speedup vs baseline: 1.6042x; 1.6042x over previous
import functools

import jax
import jax.numpy as jnp
from jax import lax
from jax.experimental import pallas as pl
from jax.experimental.pallas import tpu as pltpu
from jax.experimental.pallas import tpu_sc as plsc

B, PNUM, G, P = 8, 100, 16, 128
D = P * 2
ROWS_PER_W = 25
BATCH_PER_CORE = 4
LN2 = 0.6931471805599453
_LOG2C = (-2.5123769e-02, 1.1930088e-01, -2.7462757e-01, 4.5553029e-01,
          -7.1755898e-01, 1.4424754e+00, 2.1172477e-06)


def _ln_vec(x):
    bits = lax.bitcast_convert_type(x, jnp.int32)
    e = lax.shift_right_arithmetic(bits, jnp.full((16,), 23, jnp.int32))
    e = e - jnp.full((16,), 127, jnp.int32)
    mbits = lax.bitwise_or(
        lax.bitwise_and(bits, jnp.full((16,), 0x7FFFFF, jnp.int32)),
        jnp.full((16,), 0x3F800000, jnp.int32))
    t = lax.bitcast_convert_type(mbits, jnp.float32) - 1.0
    p = jnp.full((16,), _LOG2C[0], jnp.float32)
    for c in _LOG2C[1:]:
        p = p * t + c
    return (e.astype(jnp.float32) + p) * LN2


def _sc_body(a_hbm, gt_hbm, p_hbm, sz_hbm, out_hbm,
             a_v, g_v, chunk_v, cost_v, p_v, vec_v, part_v,
             sh_cost, sh_part):
    c = lax.axis_index("c")
    s = lax.axis_index("s")
    lb = s // BATCH_PER_CORE
    b = c * BATCH_PER_CORE + lb
    r = s % BATCH_PER_CORE
    r0 = r * ROWS_PER_W

    pltpu.sync_copy(a_hbm.at[b, pl.ds(r0, ROWS_PER_W)], a_v)
    pltpu.sync_copy(gt_hbm.at[b], g_v)

    def g2_step(e, acc):
        ge = g_v[e]
        return acc + ge * ge
    sum_g2 = lax.fori_loop(0, D, g2_step, jnp.zeros((16,), jnp.float32),
                           unroll=8)

    def row_step(i, _):
        def a2_step(k, acc):
            va = a_v[i, pl.ds(k * 16, 16)]
            return acc + va * va
        sa2 = jnp.sum(lax.fori_loop(0, D // 16, a2_step,
                                    jnp.zeros((16,), jnp.float32), unroll=4))

        def cross_step(k, acc):
            va = a_v[i, pl.ds(k * 16, 16)]
            for l in range(16):
                acc = acc + va[l] * g_v[k * 16 + l]
            return acc
        cross = lax.fori_loop(0, D // 16, cross_step,
                              jnp.zeros((16,), jnp.float32))
        chunk_v[i] = (sa2 + sum_g2 - 2.0 * cross) * (1.0 / D)
        return 0
    lax.fori_loop(0, ROWS_PER_W, row_step, 0)

    pltpu.sync_copy(chunk_v, sh_cost.at[lb, pl.ds(r0, ROWS_PER_W)])
    plsc.subcore_barrier()

    iota = lax.iota(jnp.int32, 16)

    @pl.when(r == 0)
    def _leader():
        pltpu.sync_copy(sh_cost.at[lb], cost_v)
        pltpu.sync_copy(p_hbm.at[b], p_v)
        pltpu.sync_copy(sz_hbm.at[b], vec_v)
        sz_vec = vec_v[...]
        m_m = (PNUM / sz_vec)[0]
        m_u = (PNUM / (PNUM - sz_vec))[0]

        def match_step(k, carry):
            h_acc, src_vec, colmask = carry

            def scan_row(i, bc):
                bestv, bestr = bc
                ci = cost_v[i] + colmask
                lt = ci < bestv
                bestv = jnp.where(lt, ci, bestv)
                bestr = jnp.where(lt, jnp.full((16,), i, jnp.int32), bestr)
                return bestv, bestr
            bestv, bestr = lax.fori_loop(
                0, PNUM, scan_row,
                (jnp.full((16,), 1e9, jnp.float32),
                 jnp.zeros((16,), jnp.int32)), unroll=4)

            vmin = jnp.min(bestv)
            jv = plsc.all_reduce_ffs(bestv == vmin)
            lane_j = iota == jv
            i_star = jnp.max(jnp.where(lane_j, bestr,
                                       jnp.full((16,), -1, jnp.int32)))
            src_vec = jnp.where(iota == k, jnp.full((16,), i_star, jnp.int32),
                                src_vec)
            colmask = jnp.where(lane_j, jnp.full((16,), 1e9, jnp.float32),
                                colmask)
            cost_v[i_star] = jnp.full((16,), 1e9, jnp.float32)
            return h_acc + vmin, src_vec, colmask

        h_sum, src_vec, _ = lax.fori_loop(
            0, G, match_step,
            (jnp.float32(0.0), jnp.zeros((16,), jnp.int32),
             jnp.zeros((16,), jnp.float32)))

        def l_step(k, acc):
            pv = p_v[pl.ds(k * 16, 16)]
            return acc + _ln_vec(1.0 - pv)
        sum_l = jnp.sum(lax.fori_loop(0, 128 // 16, l_step,
                                      jnp.zeros((16,), jnp.float32)))

        p_src = plsc.load_gather(p_v, [src_vec])
        sum_l_src = jnp.sum(_ln_vec(1.0 - p_src))
        pp = jnp.maximum(p_src, 0.05)
        sum_term = jnp.sum(1.0 / pp - 1.0)

        loss_b = m_u * (sum_l_src - sum_l) + m_m * sum_term
        partial = loss_b * (1.0 / (B * PNUM)) + h_sum * (0.1 / (B * G))
        vec_v[...] = jnp.where(iota == 0, jnp.full((16,), partial),
                               jnp.zeros((16,), jnp.float32))
        pltpu.sync_copy(vec_v, sh_part.at[lb])

    plsc.subcore_barrier()

    @pl.when(s == 0)
    def _core_leader():
        pltpu.sync_copy(sh_part, part_v)
        acc = part_v[0] + part_v[1] + part_v[2] + part_v[3]
        total = jnp.sum(acc)
        vec_v[...] = jnp.full((16,), total)
        pltpu.sync_copy(vec_v, out_hbm.at[c])


@jax.jit
def _run(a, gt, p_pad, sz):
    mesh = plsc.VectorSubcoreMesh(core_axis_name="c", subcore_axis_name="s")
    f = pl.kernel(
        _sc_body,
        out_type=jax.ShapeDtypeStruct((2, 16), jnp.float32),
        mesh=mesh,
        scratch_types=[
            pltpu.VMEM((ROWS_PER_W, D), jnp.float32),
            pltpu.VMEM((D, 16), jnp.float32),
            pltpu.VMEM((ROWS_PER_W, 16), jnp.float32),
            pltpu.VMEM((PNUM, 16), jnp.float32),
            pltpu.VMEM((128,), jnp.float32),
            pltpu.VMEM((16,), jnp.float32),
            pltpu.VMEM((BATCH_PER_CORE, 16), jnp.float32),
            pltpu.VMEM_SHARED((BATCH_PER_CORE, PNUM, 16), jnp.float32),
            pltpu.VMEM_SHARED((BATCH_PER_CORE, 16), jnp.float32),
        ],
        compiler_params=pltpu.CompilerParams(use_tc_tiling_on_sc=False,
                                             needs_layout_passes=False),
    )
    return f(a, gt, p_pad, sz)


def kernel(pred_poly, pred_prob, gt_py, gt_num):
    a = pred_poly.reshape(B, PNUM, D)
    gt = gt_py.reshape(B, G, D).transpose(0, 2, 1)
    p_pad = jnp.zeros((B, 128), jnp.float32).at[:, :PNUM].set(pred_prob)
    sz = jnp.broadcast_to(gt_num.astype(jnp.float32)[:, None], (B, 16))
    out = _run(a, gt, p_pad, sz)
    return out[0, 0] + out[1, 0]
